# baseline (speedup 1.0000x reference)
import functools

import numpy as np
import jax
import jax.numpy as jnp
from jax import lax
from jax.experimental import pallas as pl
from jax.experimental.pallas import tpu as pltpu
from jax.experimental.pallas import tpu_sc as plsc

N = 10000
E = 320000
D = 128
H = 128
C = 40
L = 4
ALPHA = 0.1
LAMDA = 0.5

NC = 2
NS = 16
NW = NC * NS
EPW = E // NW
KB = 80
NCH = EPW // KB
NB = 2
NRND = -(-NCH // NB)
CH = 80
NCHR = N // CH
KR = -(-NCHR // NS)

BN = 2000


def _deg_body(dst_hbm, out_hbm, didx, vbuf, zbuf, acc, dsem):
    c = lax.axis_index("c")
    s = lax.axis_index("s")
    wid = s * NC + c
    pltpu.sync_copy(dst_hbm.at[pl.ds(wid * EPW, EPW)], didx)

    ones16 = jnp.ones((16,), jnp.float32)
    zeros16 = jnp.zeros((16,), jnp.float32)

    def fill(i, carry):
        vbuf[i, :] = ones16
        return carry

    lax.fori_loop(0, KB, fill, 0)

    def zfill(i, carry):
        zbuf[i, :] = zeros16
        return carry

    lax.fori_loop(0, CH, zfill, 0)

    for k in range(KR):
        ch = s + NS * k
        @pl.when(ch < NCHR)
        def _():
            pltpu.sync_copy(zbuf, acc.at[pl.ds(ch * CH, CH)])
    plsc.subcore_barrier()

    def body(j, carry):
        pltpu.async_copy(vbuf, acc.at[didx.at[pl.ds(j * KB, KB)]], dsem, add=True)
        return carry

    lax.fori_loop(0, NCH, body, 0)

    def drain(j, carry):
        pltpu.make_async_copy(vbuf, acc.at[didx.at[pl.ds(j * KB, KB)]], dsem).wait()
        return carry

    lax.fori_loop(0, NCH, drain, 0)
    plsc.subcore_barrier()
    for k in range(KR):
        ch = s + NS * k
        @pl.when(ch < NCHR)
        def _():
            pltpu.sync_copy(acc.at[pl.ds(ch * CH, CH)], out_hbm.at[c, pl.ds(ch * CH, CH)])


_deg_kernel = pl.kernel(
    _deg_body,
    out_type=jax.ShapeDtypeStruct((NC, N, 16), jnp.float32),
    mesh=plsc.VectorSubcoreMesh(core_axis_name="c", subcore_axis_name="s"),
    scratch_types=[
        pltpu.VMEM((EPW,), jnp.int32),
        pltpu.VMEM((KB, 16), jnp.float32),
        pltpu.VMEM((CH, 16), jnp.float32),
        pltpu.VMEM_SHARED((N, 16), jnp.float32),
        pltpu.SemaphoreType.DMA,
    ],
)


def _spmm_body(g_hbm, src_hbm, dst_hbm, out_hbm, acc, sidx, didx,
               r0, r1, g0, g1, s0, s1):
    c = lax.axis_index("c")
    s = lax.axis_index("s")
    wid = s * NC + c
    rows = (r0, r1)
    gsem = (g0, g1)
    ssem = (s0, s1)

    pltpu.sync_copy(src_hbm.at[pl.ds(wid * EPW, EPW)], sidx)
    pltpu.sync_copy(dst_hbm.at[pl.ds(wid * EPW, EPW)], didx)

    zeros16 = jnp.zeros((16,), jnp.float32)

    def zfill(i, carry):
        for jj in range(H // 16):
            r0[i, pl.ds(jj * 16, 16)] = zeros16
        return carry

    lax.fori_loop(0, CH, zfill, 0)

    for k in range(KR):
        ch = s + NS * k
        @pl.when(ch < NCHR)
        def _():
            pltpu.sync_copy(r0.at[pl.ds(0, CH)], acc.at[pl.ds(ch * CH, CH)])
    plsc.subcore_barrier()

    def sch(j):
        return sidx.at[pl.ds(j * KB, KB)]

    def dch(j):
        return didx.at[pl.ds(j * KB, KB)]

    for b in range(NB):
        pltpu.async_copy(g_hbm.at[sch(b)], rows[b], gsem[b])

    def body(k, carry):
        for b in range(NB):
            ch = k * NB + b
            @pl.when(ch < NCH)
            def _():
                pltpu.make_async_copy(g_hbm.at[sch(ch)], rows[b], gsem[b]).wait()
                pltpu.async_copy(rows[b], acc.at[dch(ch)], ssem[b], add=True)
        for b in range(NB):
            ch = k * NB + b
            nch = ch + NB
            @pl.when(ch < NCH)
            def _():
                pltpu.make_async_copy(rows[b], acc.at[dch(ch)], ssem[b]).wait()
            @pl.when(nch < NCH)
            def _():
                pltpu.async_copy(g_hbm.at[sch(nch)], rows[b], gsem[b])
        return carry

    lax.fori_loop(0, NRND, body, 0)
    plsc.subcore_barrier()
    for k in range(KR):
        ch = s + NS * k
        @pl.when(ch < NCHR)
        def _():
            pltpu.sync_copy(acc.at[pl.ds(ch * CH, CH)], out_hbm.at[c, pl.ds(ch * CH, CH)])


_spmm_kernel = pl.kernel(
    _spmm_body,
    out_type=jax.ShapeDtypeStruct((NC, N, H), jnp.float32),
    mesh=plsc.VectorSubcoreMesh(core_axis_name="c", subcore_axis_name="s"),
    scratch_types=[
        pltpu.VMEM_SHARED((N, H), jnp.float32),
        pltpu.VMEM((EPW,), jnp.int32),
        pltpu.VMEM((EPW,), jnp.int32),
        pltpu.VMEM((KB, H), jnp.float32),
        pltpu.VMEM((KB, H), jnp.float32),
        pltpu.SemaphoreType.DMA,
        pltpu.SemaphoreType.DMA,
        pltpu.SemaphoreType.DMA,
        pltpu.SemaphoreType.DMA,
    ],
)


def _input_body(x_ref, w_ref, b_ref, deg_ref, h0_ref, g_ref, dinv_ref):
    deg = deg_ref[0, :, 0:1] + deg_ref[1, :, 0:1] + 1.0
    dinv = lax.rsqrt(deg)
    h = jnp.dot(x_ref[...], w_ref[...], preferred_element_type=jnp.float32)
    h = jnp.maximum(h + b_ref[...], 0.0)
    h0_ref[...] = h
    g_ref[...] = h * dinv
    dinv_ref[...] = dinv


_input_kernel = pl.pallas_call(
    _input_body,
    grid=(N // BN,),
    in_specs=[
        pl.BlockSpec((BN, D), lambda i: (i, 0)),
        pl.BlockSpec((D, H), lambda i: (0, 0)),
        pl.BlockSpec((1, H), lambda i: (0, 0)),
        pl.BlockSpec((NC, BN, 16), lambda i: (0, i, 0)),
    ],
    out_specs=[
        pl.BlockSpec((BN, H), lambda i: (i, 0)),
        pl.BlockSpec((BN, H), lambda i: (i, 0)),
        pl.BlockSpec((BN, 1), lambda i: (i, 0)),
    ],
    out_shape=[
        jax.ShapeDtypeStruct((N, H), jnp.float32),
        jax.ShapeDtypeStruct((N, H), jnp.float32),
        jax.ShapeDtypeStruct((N, 1), jnp.float32),
    ],
)


def _layer_body(s_ref, g_ref, h0_ref, dinv_ref, w_ref, out_ref):
    dinv = dinv_ref[...]
    hi = dinv * (s_ref[0] + s_ref[1] + g_ref[...])
    support = (1.0 - ALPHA) * hi + ALPHA * h0_ref[...]
    h = jnp.dot(support, w_ref[...], preferred_element_type=jnp.float32)
    out_ref[...] = jnp.maximum(h, 0.0) * dinv


_layer_kernel = pl.pallas_call(
    _layer_body,
    grid=(N // BN,),
    in_specs=[
        pl.BlockSpec((NC, BN, H), lambda i: (0, i, 0)),
        pl.BlockSpec((BN, H), lambda i: (i, 0)),
        pl.BlockSpec((BN, H), lambda i: (i, 0)),
        pl.BlockSpec((BN, 1), lambda i: (i, 0)),
        pl.BlockSpec((H, H), lambda i: (0, 0)),
    ],
    out_specs=pl.BlockSpec((BN, H), lambda i: (i, 0)),
    out_shape=jax.ShapeDtypeStruct((N, H), jnp.float32),
)


def _final_body(s_ref, g_ref, h0_ref, dinv_ref, w_ref, wo_ref, bo_ref, out_ref):
    dinv = dinv_ref[...]
    hi = dinv * (s_ref[0] + s_ref[1] + g_ref[...])
    support = (1.0 - ALPHA) * hi + ALPHA * h0_ref[...]
    h = jnp.dot(support, w_ref[...], preferred_element_type=jnp.float32)
    h = jnp.maximum(h, 0.0)
    out_ref[...] = (
        jnp.dot(h, wo_ref[...], preferred_element_type=jnp.float32) + bo_ref[...]
    )


_final_kernel = pl.pallas_call(
    _final_body,
    grid=(N // BN,),
    in_specs=[
        pl.BlockSpec((NC, BN, H), lambda i: (0, i, 0)),
        pl.BlockSpec((BN, H), lambda i: (i, 0)),
        pl.BlockSpec((BN, H), lambda i: (i, 0)),
        pl.BlockSpec((BN, 1), lambda i: (i, 0)),
        pl.BlockSpec((H, H), lambda i: (0, 0)),
        pl.BlockSpec((H, C), lambda i: (0, 0)),
        pl.BlockSpec((1, C), lambda i: (0, 0)),
    ],
    out_specs=pl.BlockSpec((BN, C), lambda i: (i, 0)),
    out_shape=jax.ShapeDtypeStruct((N, C), jnp.float32),
)


@jax.jit
def kernel(x, edge_index, W_in, b_in, Ws, W_out, b_out):
    src1 = edge_index[0]
    dst1 = edge_index[1]

    degp = _deg_kernel(dst1)
    h0, g, dinv = _input_kernel(x, W_in, b_in.reshape(1, H), degp)

    eye = jnp.eye(H, dtype=jnp.float32)
    for l in range(1, L + 1):
        theta = float(np.log(LAMDA / l + 1.0))
        Wp = theta * Ws[l - 1] + (1.0 - theta) * eye
        sp = _spmm_kernel(g, src1, dst1)
        if l < L:
            g = _layer_kernel(sp, g, h0, dinv, Wp)
        else:
            out = _final_kernel(sp, g, h0, dinv, Wp, W_out, b_out.reshape(1, C))
    return out


# speedup vs baseline: 1.2379x; 1.2379x over previous
import functools

import numpy as np
import jax
import jax.numpy as jnp
from jax import lax
from jax.experimental import pallas as pl
from jax.experimental.pallas import tpu as pltpu
from jax.experimental.pallas import tpu_sc as plsc

N = 10000
E = 320000
D = 128
H = 128
C = 40
L = 4
ALPHA = 0.1
LAMDA = 0.5

NC = 2
NS = 16
NW = NC * NS
EPW = E // NW
KB = 40
NCH = EPW // KB
NB = 5
NRND = -(-NCH // NB)
CH = 40
NCHR = N // CH
KR = -(-NCHR // NS)

BN = 2000


def _deg_body(dst_hbm, out_hbm, didx, vbuf, zbuf, acc, dsem):
    c = lax.axis_index("c")
    s = lax.axis_index("s")
    wid = s * NC + c
    pltpu.sync_copy(dst_hbm.at[pl.ds(wid * EPW, EPW)], didx)

    ones16 = jnp.ones((16,), jnp.float32)
    zeros16 = jnp.zeros((16,), jnp.float32)

    def fill(i, carry):
        vbuf[i, :] = ones16
        return carry

    lax.fori_loop(0, KB, fill, 0)

    def zfill(i, carry):
        zbuf[i, :] = zeros16
        return carry

    lax.fori_loop(0, CH, zfill, 0)

    for k in range(KR):
        ch = s + NS * k
        @pl.when(ch < NCHR)
        def _():
            pltpu.sync_copy(zbuf, acc.at[pl.ds(ch * CH, CH)])
    plsc.subcore_barrier()

    def body(j, carry):
        pltpu.async_copy(vbuf, acc.at[didx.at[pl.ds(j * KB, KB)]], dsem, add=True)
        return carry

    lax.fori_loop(0, NCH, body, 0)

    def drain(j, carry):
        pltpu.make_async_copy(vbuf, acc.at[didx.at[pl.ds(j * KB, KB)]], dsem).wait()
        return carry

    lax.fori_loop(0, NCH, drain, 0)
    plsc.subcore_barrier()
    for k in range(KR):
        ch = s + NS * k
        @pl.when(ch < NCHR)
        def _():
            pltpu.sync_copy(acc.at[pl.ds(ch * CH, CH)], out_hbm.at[c, pl.ds(ch * CH, CH)])


_deg_kernel = pl.kernel(
    _deg_body,
    out_type=jax.ShapeDtypeStruct((NC, N, 16), jnp.float32),
    mesh=plsc.VectorSubcoreMesh(core_axis_name="c", subcore_axis_name="s"),
    scratch_types=[
        pltpu.VMEM((EPW,), jnp.int32),
        pltpu.VMEM((KB, 16), jnp.float32),
        pltpu.VMEM((CH, 16), jnp.float32),
        pltpu.VMEM_SHARED((N, 16), jnp.float32),
        pltpu.SemaphoreType.DMA,
    ],
)


def _spmm_body(g_hbm, src_hbm, dst_hbm, out_hbm, acc, sidx, didx,
               r0, r1, r2, r3, r4, g0, g1, g2, g3, g4, s0, s1, s2, s3, s4):
    c = lax.axis_index("c")
    s = lax.axis_index("s")
    wid = s * NC + c
    rows = (r0, r1, r2, r3, r4)
    gsem = (g0, g1, g2, g3, g4)
    ssem = (s0, s1, s2, s3, s4)

    pltpu.sync_copy(src_hbm.at[pl.ds(wid * EPW, EPW)], sidx)
    pltpu.sync_copy(dst_hbm.at[pl.ds(wid * EPW, EPW)], didx)

    zeros16 = jnp.zeros((16,), jnp.float32)

    def zfill(i, carry):
        for jj in range(H // 16):
            r0[i, pl.ds(jj * 16, 16)] = zeros16
        return carry

    lax.fori_loop(0, CH, zfill, 0)

    for k in range(KR):
        ch = s + NS * k
        @pl.when(ch < NCHR)
        def _():
            pltpu.sync_copy(r0.at[pl.ds(0, CH)], acc.at[pl.ds(ch * CH, CH)])
    plsc.subcore_barrier()

    def sch(j):
        return sidx.at[pl.ds(j * KB, KB)]

    def dch(j):
        return didx.at[pl.ds(j * KB, KB)]

    for b in range(NB):
        pltpu.async_copy(g_hbm.at[sch(b)], rows[b], gsem[b])

    def body(k, carry):
        for b in range(NB):
            ch = k * NB + b
            @pl.when(ch < NCH)
            def _():
                pltpu.make_async_copy(g_hbm.at[sch(ch)], rows[b], gsem[b]).wait()
                pltpu.async_copy(rows[b], acc.at[dch(ch)], ssem[b], add=True)
        for b in range(NB):
            ch = k * NB + b
            nch = ch + NB
            @pl.when(ch < NCH)
            def _():
                pltpu.make_async_copy(rows[b], acc.at[dch(ch)], ssem[b]).wait()
            @pl.when(nch < NCH)
            def _():
                pltpu.async_copy(g_hbm.at[sch(nch)], rows[b], gsem[b])
        return carry

    lax.fori_loop(0, NRND, body, 0)
    plsc.subcore_barrier()
    for k in range(KR):
        ch = s + NS * k
        @pl.when(ch < NCHR)
        def _():
            pltpu.sync_copy(acc.at[pl.ds(ch * CH, CH)], out_hbm.at[c, pl.ds(ch * CH, CH)])


_spmm_kernel = pl.kernel(
    _spmm_body,
    out_type=jax.ShapeDtypeStruct((NC, N, H), jnp.float32),
    mesh=plsc.VectorSubcoreMesh(core_axis_name="c", subcore_axis_name="s"),
    scratch_types=[
        pltpu.VMEM_SHARED((N, H), jnp.float32),
        pltpu.VMEM((EPW,), jnp.int32),
        pltpu.VMEM((EPW,), jnp.int32),
        pltpu.VMEM((KB, H), jnp.float32),
        pltpu.VMEM((KB, H), jnp.float32),
        pltpu.VMEM((KB, H), jnp.float32),
        pltpu.VMEM((KB, H), jnp.float32),
        pltpu.VMEM((KB, H), jnp.float32),
        pltpu.SemaphoreType.DMA,
        pltpu.SemaphoreType.DMA,
        pltpu.SemaphoreType.DMA,
        pltpu.SemaphoreType.DMA,
        pltpu.SemaphoreType.DMA,
        pltpu.SemaphoreType.DMA,
        pltpu.SemaphoreType.DMA,
        pltpu.SemaphoreType.DMA,
        pltpu.SemaphoreType.DMA,
        pltpu.SemaphoreType.DMA,
    ],
)


def _input_body(x_ref, w_ref, b_ref, deg_ref, h0_ref, g_ref, dinv_ref):
    deg = deg_ref[0, :, 0:1] + deg_ref[1, :, 0:1] + 1.0
    dinv = lax.rsqrt(deg)
    h = jnp.dot(x_ref[...], w_ref[...], preferred_element_type=jnp.float32)
    h = jnp.maximum(h + b_ref[...], 0.0)
    h0_ref[...] = h
    g_ref[...] = h * dinv
    dinv_ref[...] = dinv


_input_kernel = pl.pallas_call(
    _input_body,
    grid=(N // BN,),
    in_specs=[
        pl.BlockSpec((BN, D), lambda i: (i, 0)),
        pl.BlockSpec((D, H), lambda i: (0, 0)),
        pl.BlockSpec((1, H), lambda i: (0, 0)),
        pl.BlockSpec((NC, BN, 16), lambda i: (0, i, 0)),
    ],
    out_specs=[
        pl.BlockSpec((BN, H), lambda i: (i, 0)),
        pl.BlockSpec((BN, H), lambda i: (i, 0)),
        pl.BlockSpec((BN, 1), lambda i: (i, 0)),
    ],
    out_shape=[
        jax.ShapeDtypeStruct((N, H), jnp.float32),
        jax.ShapeDtypeStruct((N, H), jnp.float32),
        jax.ShapeDtypeStruct((N, 1), jnp.float32),
    ],
)


def _layer_body(s_ref, g_ref, h0_ref, dinv_ref, w_ref, out_ref):
    dinv = dinv_ref[...]
    hi = dinv * (s_ref[0] + s_ref[1] + g_ref[...])
    support = (1.0 - ALPHA) * hi + ALPHA * h0_ref[...]
    h = jnp.dot(support, w_ref[...], preferred_element_type=jnp.float32)
    out_ref[...] = jnp.maximum(h, 0.0) * dinv


_layer_kernel = pl.pallas_call(
    _layer_body,
    grid=(N // BN,),
    in_specs=[
        pl.BlockSpec((NC, BN, H), lambda i: (0, i, 0)),
        pl.BlockSpec((BN, H), lambda i: (i, 0)),
        pl.BlockSpec((BN, H), lambda i: (i, 0)),
        pl.BlockSpec((BN, 1), lambda i: (i, 0)),
        pl.BlockSpec((H, H), lambda i: (0, 0)),
    ],
    out_specs=pl.BlockSpec((BN, H), lambda i: (i, 0)),
    out_shape=jax.ShapeDtypeStruct((N, H), jnp.float32),
)


def _final_body(s_ref, g_ref, h0_ref, dinv_ref, w_ref, wo_ref, bo_ref, out_ref):
    dinv = dinv_ref[...]
    hi = dinv * (s_ref[0] + s_ref[1] + g_ref[...])
    support = (1.0 - ALPHA) * hi + ALPHA * h0_ref[...]
    h = jnp.dot(support, w_ref[...], preferred_element_type=jnp.float32)
    h = jnp.maximum(h, 0.0)
    out_ref[...] = (
        jnp.dot(h, wo_ref[...], preferred_element_type=jnp.float32) + bo_ref[...]
    )


_final_kernel = pl.pallas_call(
    _final_body,
    grid=(N // BN,),
    in_specs=[
        pl.BlockSpec((NC, BN, H), lambda i: (0, i, 0)),
        pl.BlockSpec((BN, H), lambda i: (i, 0)),
        pl.BlockSpec((BN, H), lambda i: (i, 0)),
        pl.BlockSpec((BN, 1), lambda i: (i, 0)),
        pl.BlockSpec((H, H), lambda i: (0, 0)),
        pl.BlockSpec((H, C), lambda i: (0, 0)),
        pl.BlockSpec((1, C), lambda i: (0, 0)),
    ],
    out_specs=pl.BlockSpec((BN, C), lambda i: (i, 0)),
    out_shape=jax.ShapeDtypeStruct((N, C), jnp.float32),
)


@jax.jit
def kernel(x, edge_index, W_in, b_in, Ws, W_out, b_out):
    src1 = edge_index[0]
    dst1 = edge_index[1]

    degp = _deg_kernel(dst1)
    h0, g, dinv = _input_kernel(x, W_in, b_in.reshape(1, H), degp)

    eye = jnp.eye(H, dtype=jnp.float32)
    for l in range(1, L + 1):
        theta = float(np.log(LAMDA / l + 1.0))
        Wp = theta * Ws[l - 1] + (1.0 - theta) * eye
        sp = _spmm_kernel(g, src1, dst1)
        if l < L:
            g = _layer_kernel(sp, g, h0, dinv, Wp)
        else:
            out = _final_kernel(sp, g, h0, dinv, Wp, W_out, b_out.reshape(1, C))
    return out


# speedup vs baseline: 1.2724x; 1.0279x over previous
import functools

import numpy as np
import jax
import jax.numpy as jnp
from jax import lax
from jax.experimental import pallas as pl
from jax.experimental.pallas import tpu as pltpu
from jax.experimental.pallas import tpu_sc as plsc

N = 10000
E = 320000
D = 128
H = 128
C = 40
L = 4
ALPHA = 0.1
LAMDA = 0.5

NC = 2
NS = 16
NW = NC * NS
EPW = E // NW
KB = 40
NCH = EPW // KB
NB = 8
NRND = -(-NCH // NB)
PK = NB * KB
EPAD = NRND * PK
CH = 40
NCHR = N // CH
KR = -(-NCHR // NS)

BN = 2000


def _deg_body(dst_hbm, out_hbm, didx, vbuf, zbuf, acc, dsem):
    c = lax.axis_index("c")
    s = lax.axis_index("s")
    wid = s * NC + c
    pltpu.sync_copy(dst_hbm.at[pl.ds(wid * EPW, EPW)], didx)

    ones16 = jnp.ones((16,), jnp.float32)
    zeros16 = jnp.zeros((16,), jnp.float32)

    def fill(i, carry):
        vbuf[i, :] = ones16
        return carry

    lax.fori_loop(0, KB, fill, 0)

    def zfill(i, carry):
        zbuf[i, :] = zeros16
        return carry

    lax.fori_loop(0, CH, zfill, 0)

    for k in range(KR):
        ch = s + NS * k
        @pl.when(ch < NCHR)
        def _():
            pltpu.sync_copy(zbuf, acc.at[pl.ds(ch * CH, CH)])
    plsc.subcore_barrier()

    def body(j, carry):
        pltpu.async_copy(vbuf, acc.at[didx.at[pl.ds(j * KB, KB)]], dsem, add=True)
        return carry

    lax.fori_loop(0, NCH, body, 0)

    def drain(j, carry):
        pltpu.make_async_copy(vbuf, acc.at[didx.at[pl.ds(j * KB, KB)]], dsem).wait()
        return carry

    lax.fori_loop(0, NCH, drain, 0)
    plsc.subcore_barrier()
    for k in range(KR):
        ch = s + NS * k
        @pl.when(ch < NCHR)
        def _():
            pltpu.sync_copy(acc.at[pl.ds(ch * CH, CH)], out_hbm.at[c, pl.ds(ch * CH, CH)])


_deg_kernel = pl.kernel(
    _deg_body,
    out_type=jax.ShapeDtypeStruct((NC, N, 16), jnp.float32),
    mesh=plsc.VectorSubcoreMesh(core_axis_name="c", subcore_axis_name="s"),
    scratch_types=[
        pltpu.VMEM((EPW,), jnp.int32),
        pltpu.VMEM((KB, 16), jnp.float32),
        pltpu.VMEM((CH, 16), jnp.float32),
        pltpu.VMEM_SHARED((N, 16), jnp.float32),
        pltpu.SemaphoreType.DMA,
    ],
)


def _spmm_body(g_hbm, pk_hbm, out_hbm, acc, i0, i1,
               r0, r1, r2, r3, r4, r5, r6, r7,
               is0, is1,
               g0, g1, g2, g3, g4, g5, g6, g7,
               s0, s1, s2, s3, s4, s5, s6, s7):
    c = lax.axis_index("c")
    s = lax.axis_index("s")
    wid = s * NC + c
    rows = (r0, r1, r2, r3, r4, r5, r6, r7)
    gsem = (g0, g1, g2, g3, g4, g5, g6, g7)
    ssem = (s0, s1, s2, s3, s4, s5, s6, s7)
    ibuf = (i0, i1)
    isem = (is0, is1)

    zeros16 = jnp.zeros((16,), jnp.float32)

    def zfill(i, carry):
        for jj in range(H // 16):
            r0[i, pl.ds(jj * 16, 16)] = zeros16
        return carry

    lax.fori_loop(0, CH, zfill, 0)

    for k in range(KR):
        ch = s + NS * k
        @pl.when(ch < NCHR)
        def _():
            pltpu.sync_copy(r0.at[pl.ds(0, CH)], acc.at[pl.ds(ch * CH, CH)])
    plsc.subcore_barrier()

    def blk_copy(k, p):
        return pltpu.make_async_copy(
            pk_hbm.at[pl.ds((wid * NRND + k) * (2 * PK), 2 * PK)], ibuf[p], isem[p])

    def sch(p, b):
        return ibuf[p].at[pl.ds(b * KB, KB)]

    def dch(p, b):
        return ibuf[p].at[pl.ds(PK + b * KB, KB)]

    blk_copy(0, 0).start()
    blk_copy(1, 1).start()
    blk_copy(0, 0).wait()
    for b in range(NB):
        pltpu.async_copy(g_hbm.at[sch(0, b)], rows[b], gsem[b])

    def round_(k, kk, p):
        for b in range(NB):
            ch = k * NB + b
            @pl.when(ch < NCH)
            def _():
                pltpu.make_async_copy(g_hbm.at[sch(p, b)], rows[b], gsem[b]).wait()
                pltpu.async_copy(rows[b], acc.at[dch(p, b)], ssem[b], add=True)
        @pl.when(k + 1 < NRND)
        def _():
            blk_copy(k + 1, 1 - p).wait()
        for b in range(NB):
            ch = k * NB + b
            nch = ch + NB
            @pl.when(ch < NCH)
            def _():
                pltpu.make_async_copy(rows[b], acc.at[dch(p, b)], ssem[b]).wait()
            @pl.when(nch < NCH)
            def _():
                pltpu.async_copy(g_hbm.at[sch(1 - p, b)], rows[b], gsem[b])
        @pl.when(k + 2 < NRND)
        def _():
            blk_copy(k + 2, p).start()

    def body(kk, carry):
        round_(2 * kk, kk, 0)
        round_(2 * kk + 1, kk, 1)
        return carry

    lax.fori_loop(0, NRND // 2, body, 0)
    plsc.subcore_barrier()
    for k in range(KR):
        ch = s + NS * k
        @pl.when(ch < NCHR)
        def _():
            pltpu.sync_copy(acc.at[pl.ds(ch * CH, CH)], out_hbm.at[c, pl.ds(ch * CH, CH)])


_spmm_kernel = pl.kernel(
    _spmm_body,
    out_type=jax.ShapeDtypeStruct((NC, N, H), jnp.float32),
    mesh=plsc.VectorSubcoreMesh(core_axis_name="c", subcore_axis_name="s"),
    scratch_types=(
        [pltpu.VMEM_SHARED((N, H), jnp.float32)]
        + [pltpu.VMEM((2 * PK,), jnp.int32) for _ in range(2)]
        + [pltpu.VMEM((KB, H), jnp.float32) for _ in range(NB)]
        + [pltpu.SemaphoreType.DMA for _ in range(2 + 2 * NB)]
    ),
)


def _input_body(x_ref, w_ref, b_ref, deg_ref, h0_ref, g_ref, dinv_ref):
    deg = deg_ref[0, :, 0:1] + deg_ref[1, :, 0:1] + 1.0
    dinv = lax.rsqrt(deg)
    h = jnp.dot(x_ref[...], w_ref[...], preferred_element_type=jnp.float32)
    h = jnp.maximum(h + b_ref[...], 0.0)
    h0_ref[...] = h
    g_ref[...] = h * dinv
    dinv_ref[...] = dinv


_input_kernel = pl.pallas_call(
    _input_body,
    grid=(N // BN,),
    in_specs=[
        pl.BlockSpec((BN, D), lambda i: (i, 0)),
        pl.BlockSpec((D, H), lambda i: (0, 0)),
        pl.BlockSpec((1, H), lambda i: (0, 0)),
        pl.BlockSpec((NC, BN, 16), lambda i: (0, i, 0)),
    ],
    out_specs=[
        pl.BlockSpec((BN, H), lambda i: (i, 0)),
        pl.BlockSpec((BN, H), lambda i: (i, 0)),
        pl.BlockSpec((BN, 1), lambda i: (i, 0)),
    ],
    out_shape=[
        jax.ShapeDtypeStruct((N, H), jnp.float32),
        jax.ShapeDtypeStruct((N, H), jnp.float32),
        jax.ShapeDtypeStruct((N, 1), jnp.float32),
    ],
)


def _layer_body(s_ref, g_ref, h0_ref, dinv_ref, w_ref, out_ref):
    dinv = dinv_ref[...]
    hi = dinv * (s_ref[0] + s_ref[1] + g_ref[...])
    support = (1.0 - ALPHA) * hi + ALPHA * h0_ref[...]
    h = jnp.dot(support, w_ref[...], preferred_element_type=jnp.float32)
    out_ref[...] = jnp.maximum(h, 0.0) * dinv


_layer_kernel = pl.pallas_call(
    _layer_body,
    grid=(N // BN,),
    in_specs=[
        pl.BlockSpec((NC, BN, H), lambda i: (0, i, 0)),
        pl.BlockSpec((BN, H), lambda i: (i, 0)),
        pl.BlockSpec((BN, H), lambda i: (i, 0)),
        pl.BlockSpec((BN, 1), lambda i: (i, 0)),
        pl.BlockSpec((H, H), lambda i: (0, 0)),
    ],
    out_specs=pl.BlockSpec((BN, H), lambda i: (i, 0)),
    out_shape=jax.ShapeDtypeStruct((N, H), jnp.float32),
)


def _final_body(s_ref, g_ref, h0_ref, dinv_ref, w_ref, wo_ref, bo_ref, out_ref):
    dinv = dinv_ref[...]
    hi = dinv * (s_ref[0] + s_ref[1] + g_ref[...])
    support = (1.0 - ALPHA) * hi + ALPHA * h0_ref[...]
    h = jnp.dot(support, w_ref[...], preferred_element_type=jnp.float32)
    h = jnp.maximum(h, 0.0)
    out_ref[...] = (
        jnp.dot(h, wo_ref[...], preferred_element_type=jnp.float32) + bo_ref[...]
    )


_final_kernel = pl.pallas_call(
    _final_body,
    grid=(N // BN,),
    in_specs=[
        pl.BlockSpec((NC, BN, H), lambda i: (0, i, 0)),
        pl.BlockSpec((BN, H), lambda i: (i, 0)),
        pl.BlockSpec((BN, H), lambda i: (i, 0)),
        pl.BlockSpec((BN, 1), lambda i: (i, 0)),
        pl.BlockSpec((H, H), lambda i: (0, 0)),
        pl.BlockSpec((H, C), lambda i: (0, 0)),
        pl.BlockSpec((1, C), lambda i: (0, 0)),
    ],
    out_specs=pl.BlockSpec((BN, C), lambda i: (i, 0)),
    out_shape=jax.ShapeDtypeStruct((N, C), jnp.float32),
)


@jax.jit
def kernel(x, edge_index, W_in, b_in, Ws, W_out, b_out):
    src1 = edge_index[0]
    dst1 = edge_index[1]
    pad = jnp.zeros((NW, EPAD - EPW), jnp.int32)
    sblk = jnp.concatenate([src1.reshape(NW, EPW), pad], axis=1).reshape(NW, NRND, 1, PK)
    dblk = jnp.concatenate([dst1.reshape(NW, EPW), pad], axis=1).reshape(NW, NRND, 1, PK)
    packed = jnp.concatenate([sblk, dblk], axis=2).reshape(-1)

    degp = _deg_kernel(dst1)
    h0, g, dinv = _input_kernel(x, W_in, b_in.reshape(1, H), degp)

    eye = jnp.eye(H, dtype=jnp.float32)
    for l in range(1, L + 1):
        theta = float(np.log(LAMDA / l + 1.0))
        Wp = theta * Ws[l - 1] + (1.0 - theta) * eye
        sp = _spmm_kernel(g, packed)
        if l < L:
            g = _layer_kernel(sp, g, h0, dinv, Wp)
        else:
            out = _final_kernel(sp, g, h0, dinv, Wp, W_out, b_out.reshape(1, C))
    return out


# speedup vs baseline: 1.3146x; 1.0331x over previous
import functools

import numpy as np
import jax
import jax.numpy as jnp
from jax import lax
from jax.experimental import pallas as pl
from jax.experimental.pallas import tpu as pltpu
from jax.experimental.pallas import tpu_sc as plsc

N = 10000
E = 320000
D = 128
H = 128
C = 40
L = 4
ALPHA = 0.1
LAMDA = 0.5

NC = 2
NS = 16
NW = NC * NS
EPW = E // NW
KB = 40
NCH = EPW // KB
NB = 8
NRND = -(-NCH // NB)
PK = NB * KB
EPAD = NRND * PK
CH = 40
NCHR = N // CH
KR = -(-NCHR // NS)

BN = 2000


def _deg_body(dst_hbm, out_hbm, didx, vbuf, zbuf, acc, dsem):
    c = lax.axis_index("c")
    s = lax.axis_index("s")
    wid = s * NC + c
    pltpu.sync_copy(dst_hbm.at[pl.ds(wid * EPW, EPW)], didx)

    ones16 = jnp.ones((16,), jnp.float32)
    zeros16 = jnp.zeros((16,), jnp.float32)

    def fill(i, carry):
        vbuf[i, :] = ones16
        return carry

    lax.fori_loop(0, KB, fill, 0)

    def zfill(i, carry):
        zbuf[i, :] = zeros16
        return carry

    lax.fori_loop(0, CH, zfill, 0)

    for k in range(KR):
        ch = s + NS * k
        @pl.when(ch < NCHR)
        def _():
            pltpu.async_copy(zbuf, acc.at[pl.ds(ch * CH, CH)], dsem)
    for k in range(KR):
        ch = s + NS * k
        @pl.when(ch < NCHR)
        def _():
            pltpu.make_async_copy(zbuf, acc.at[pl.ds(ch * CH, CH)], dsem).wait()
    plsc.subcore_barrier()

    def body(j, carry):
        pltpu.async_copy(vbuf, acc.at[didx.at[pl.ds(j * KB, KB)]], dsem, add=True)
        return carry

    lax.fori_loop(0, NCH, body, 0)

    def drain(j, carry):
        pltpu.make_async_copy(vbuf, acc.at[didx.at[pl.ds(j * KB, KB)]], dsem).wait()
        return carry

    lax.fori_loop(0, NCH, drain, 0)
    plsc.subcore_barrier()
    for k in range(KR):
        ch = s + NS * k
        @pl.when(ch < NCHR)
        def _():
            pltpu.async_copy(acc.at[pl.ds(ch * CH, CH)], out_hbm.at[c, pl.ds(ch * CH, CH)], dsem)
    for k in range(KR):
        ch = s + NS * k
        @pl.when(ch < NCHR)
        def _():
            pltpu.make_async_copy(acc.at[pl.ds(ch * CH, CH)], out_hbm.at[c, pl.ds(ch * CH, CH)], dsem).wait()


_deg_kernel = pl.kernel(
    _deg_body,
    out_type=jax.ShapeDtypeStruct((NC, N, 16), jnp.float32),
    mesh=plsc.VectorSubcoreMesh(core_axis_name="c", subcore_axis_name="s"),
    scratch_types=[
        pltpu.VMEM((EPW,), jnp.int32),
        pltpu.VMEM((KB, 16), jnp.float32),
        pltpu.VMEM((CH, 16), jnp.float32),
        pltpu.VMEM_SHARED((N, 16), jnp.float32),
        pltpu.SemaphoreType.DMA,
    ],
)


def _spmm_body(g_hbm, pk_hbm, out_hbm, acc, i0, i1,
               r0, r1, r2, r3, r4, r5, r6, r7,
               is0, is1,
               g0, g1, g2, g3, g4, g5, g6, g7,
               s0, s1, s2, s3, s4, s5, s6, s7):
    c = lax.axis_index("c")
    s = lax.axis_index("s")
    wid = s * NC + c
    rows = (r0, r1, r2, r3, r4, r5, r6, r7)
    gsem = (g0, g1, g2, g3, g4, g5, g6, g7)
    ssem = (s0, s1, s2, s3, s4, s5, s6, s7)
    ibuf = (i0, i1)
    isem = (is0, is1)

    zeros16 = jnp.zeros((16,), jnp.float32)

    def zfill(i, carry):
        for jj in range(H // 16):
            r0[i, pl.ds(jj * 16, 16)] = zeros16
        return carry

    lax.fori_loop(0, CH, zfill, 0)

    for k in range(KR):
        ch = s + NS * k
        @pl.when(ch < NCHR)
        def _():
            pltpu.async_copy(r0.at[pl.ds(0, CH)], acc.at[pl.ds(ch * CH, CH)], is0)
    for k in range(KR):
        ch = s + NS * k
        @pl.when(ch < NCHR)
        def _():
            pltpu.make_async_copy(r0.at[pl.ds(0, CH)], acc.at[pl.ds(ch * CH, CH)], is0).wait()
    plsc.subcore_barrier()

    def blk_copy(k, p):
        return pltpu.make_async_copy(
            pk_hbm.at[pl.ds((wid * NRND + k) * (2 * PK), 2 * PK)], ibuf[p], isem[p])

    def sch(p, b):
        return ibuf[p].at[pl.ds(b * KB, KB)]

    def dch(p, b):
        return ibuf[p].at[pl.ds(PK + b * KB, KB)]

    blk_copy(0, 0).start()
    blk_copy(1, 1).start()
    blk_copy(0, 0).wait()
    for b in range(NB):
        pltpu.async_copy(g_hbm.at[sch(0, b)], rows[b], gsem[b])

    def round_(k, kk, p):
        for b in range(NB):
            ch = k * NB + b
            @pl.when(ch < NCH)
            def _():
                pltpu.make_async_copy(g_hbm.at[sch(p, b)], rows[b], gsem[b]).wait()
                pltpu.async_copy(rows[b], acc.at[dch(p, b)], ssem[b], add=True)
        @pl.when(k + 1 < NRND)
        def _():
            blk_copy(k + 1, 1 - p).wait()
        for b in range(NB):
            ch = k * NB + b
            nch = ch + NB
            @pl.when(ch < NCH)
            def _():
                pltpu.make_async_copy(rows[b], acc.at[dch(p, b)], ssem[b]).wait()
            @pl.when(nch < NCH)
            def _():
                pltpu.async_copy(g_hbm.at[sch(1 - p, b)], rows[b], gsem[b])
        @pl.when(k + 2 < NRND)
        def _():
            blk_copy(k + 2, p).start()

    def body(kk, carry):
        round_(2 * kk, kk, 0)
        round_(2 * kk + 1, kk, 1)
        return carry

    lax.fori_loop(0, NRND // 2, body, 0)
    plsc.subcore_barrier()
    for k in range(KR):
        ch = s + NS * k
        @pl.when(ch < NCHR)
        def _():
            pltpu.async_copy(acc.at[pl.ds(ch * CH, CH)], out_hbm.at[c, pl.ds(ch * CH, CH)], is0)
    for k in range(KR):
        ch = s + NS * k
        @pl.when(ch < NCHR)
        def _():
            pltpu.make_async_copy(acc.at[pl.ds(ch * CH, CH)], out_hbm.at[c, pl.ds(ch * CH, CH)], is0).wait()


_spmm_kernel = pl.kernel(
    _spmm_body,
    out_type=jax.ShapeDtypeStruct((NC, N, H), jnp.float32),
    mesh=plsc.VectorSubcoreMesh(core_axis_name="c", subcore_axis_name="s"),
    scratch_types=(
        [pltpu.VMEM_SHARED((N, H), jnp.float32)]
        + [pltpu.VMEM((2 * PK,), jnp.int32) for _ in range(2)]
        + [pltpu.VMEM((KB, H), jnp.float32) for _ in range(NB)]
        + [pltpu.SemaphoreType.DMA for _ in range(2 + 2 * NB)]
    ),
)


def _input_body(x_ref, w_ref, b_ref, h0_ref):
    h = jnp.dot(x_ref[...], w_ref[...], preferred_element_type=jnp.float32)
    h0_ref[...] = jnp.maximum(h + b_ref[...], 0.0)


_input_kernel = pl.pallas_call(
    _input_body,
    grid=(N // BN,),
    in_specs=[
        pl.BlockSpec((BN, D), lambda i: (i, 0)),
        pl.BlockSpec((D, H), lambda i: (0, 0)),
        pl.BlockSpec((1, H), lambda i: (0, 0)),
    ],
    out_specs=pl.BlockSpec((BN, H), lambda i: (i, 0)),
    out_shape=jax.ShapeDtypeStruct((N, H), jnp.float32),
)


def _prep_body(h0_ref, deg_ref, g_ref, dinv_ref):
    deg = deg_ref[0, :, 0:1] + deg_ref[1, :, 0:1] + 1.0
    dinv = lax.rsqrt(deg)
    g_ref[...] = h0_ref[...] * dinv
    dinv_ref[...] = dinv


_prep_kernel = pl.pallas_call(
    _prep_body,
    grid=(N // BN,),
    in_specs=[
        pl.BlockSpec((BN, H), lambda i: (i, 0)),
        pl.BlockSpec((NC, BN, 16), lambda i: (0, i, 0)),
    ],
    out_specs=[
        pl.BlockSpec((BN, H), lambda i: (i, 0)),
        pl.BlockSpec((BN, 1), lambda i: (i, 0)),
    ],
    out_shape=[
        jax.ShapeDtypeStruct((N, H), jnp.float32),
        jax.ShapeDtypeStruct((N, 1), jnp.float32),
    ],
)


def _layer_body(s_ref, g_ref, h0_ref, dinv_ref, w_ref, out_ref):
    dinv = dinv_ref[...]
    hi = dinv * (s_ref[0] + s_ref[1] + g_ref[...])
    support = (1.0 - ALPHA) * hi + ALPHA * h0_ref[...]
    h = jnp.dot(support, w_ref[...], preferred_element_type=jnp.float32)
    out_ref[...] = jnp.maximum(h, 0.0) * dinv


_layer_kernel = pl.pallas_call(
    _layer_body,
    grid=(N // BN,),
    in_specs=[
        pl.BlockSpec((NC, BN, H), lambda i: (0, i, 0)),
        pl.BlockSpec((BN, H), lambda i: (i, 0)),
        pl.BlockSpec((BN, H), lambda i: (i, 0)),
        pl.BlockSpec((BN, 1), lambda i: (i, 0)),
        pl.BlockSpec((H, H), lambda i: (0, 0)),
    ],
    out_specs=pl.BlockSpec((BN, H), lambda i: (i, 0)),
    out_shape=jax.ShapeDtypeStruct((N, H), jnp.float32),
)


def _final_body(s_ref, g_ref, h0_ref, dinv_ref, w_ref, wo_ref, bo_ref, out_ref):
    dinv = dinv_ref[...]
    hi = dinv * (s_ref[0] + s_ref[1] + g_ref[...])
    support = (1.0 - ALPHA) * hi + ALPHA * h0_ref[...]
    h = jnp.dot(support, w_ref[...], preferred_element_type=jnp.float32)
    h = jnp.maximum(h, 0.0)
    out_ref[...] = (
        jnp.dot(h, wo_ref[...], preferred_element_type=jnp.float32) + bo_ref[...]
    )


_final_kernel = pl.pallas_call(
    _final_body,
    grid=(N // BN,),
    in_specs=[
        pl.BlockSpec((NC, BN, H), lambda i: (0, i, 0)),
        pl.BlockSpec((BN, H), lambda i: (i, 0)),
        pl.BlockSpec((BN, H), lambda i: (i, 0)),
        pl.BlockSpec((BN, 1), lambda i: (i, 0)),
        pl.BlockSpec((H, H), lambda i: (0, 0)),
        pl.BlockSpec((H, C), lambda i: (0, 0)),
        pl.BlockSpec((1, C), lambda i: (0, 0)),
    ],
    out_specs=pl.BlockSpec((BN, C), lambda i: (i, 0)),
    out_shape=jax.ShapeDtypeStruct((N, C), jnp.float32),
)


@jax.jit
def kernel(x, edge_index, W_in, b_in, Ws, W_out, b_out):
    src1 = edge_index[0]
    dst1 = edge_index[1]
    pad = jnp.zeros((NW, EPAD - EPW), jnp.int32)
    sblk = jnp.concatenate([src1.reshape(NW, EPW), pad], axis=1).reshape(NW, NRND, 1, PK)
    dblk = jnp.concatenate([dst1.reshape(NW, EPW), pad], axis=1).reshape(NW, NRND, 1, PK)
    packed = jnp.concatenate([sblk, dblk], axis=2).reshape(-1)

    degp = _deg_kernel(dst1)
    h0 = _input_kernel(x, W_in, b_in.reshape(1, H))
    g, dinv = _prep_kernel(h0, degp)

    eye = jnp.eye(H, dtype=jnp.float32)
    for l in range(1, L + 1):
        theta = float(np.log(LAMDA / l + 1.0))
        Wp = theta * Ws[l - 1] + (1.0 - theta) * eye
        sp = _spmm_kernel(g, packed)
        if l < L:
            g = _layer_kernel(sp, g, h0, dinv, Wp)
        else:
            out = _final_kernel(sp, g, h0, dinv, Wp, W_out, b_out.reshape(1, C))
    return out
